# baseline (device time: 37775 ns/iter reference)
import jax
import jax.numpy as jnp
from jax import lax
from jax.experimental import pallas as pl
from jax.experimental.pallas import tpu as pltpu

N_DEV = 16
N_IDX = 1024
D = 512
V_PER = 4096
CHUNK = N_IDX // N_DEV


def kernel(table, idx):
    idx2 = idx.reshape(N_DEV, CHUNK)

    def body(idx_ref, table_ref, out_ref, tblb_ref, p0_ref, p1_ref, red_ref,
             p2_ref, p1_send, p1_recv, p2_send, p2_recv):
        my = lax.axis_index("i")

        barrier_sem = pltpu.get_barrier_semaphore()
        for o in range(1, N_DEV):
            pl.semaphore_signal(
                barrier_sem, inc=1,
                device_id=(jnp.mod(my + o, N_DEV),),
                device_id_type=pl.DeviceIdType.MESH,
            )
        pl.semaphore_wait(barrier_sem, N_DEV - 1)

        tblb_ref[...] = table_ref[...].astype(jnp.bfloat16)

        base = my * V_PER
        col = lax.broadcasted_iota(jnp.int32, (CHUNK, V_PER), 1)

        def gather_chunk(c, slot):
            idxs = idx_ref[pl.ds(c, 1), :].reshape(CHUNK, 1)
            loc = idxs - base
            valid = jnp.logical_and(loc >= 0, loc < V_PER)
            oh = jnp.logical_and(col == loc, valid).astype(jnp.bfloat16)
            part = jnp.dot(oh, tblb_ref[...],
                           preferred_element_type=jnp.float32)
            p0_ref[slot] = part.astype(jnp.bfloat16)

        p1_rdmas = []
        for o in range(1, N_DEV):
            e = jnp.mod(my + o, N_DEV)
            gather_chunk(e, o)
            rdma = pltpu.make_async_remote_copy(
                src_ref=p0_ref.at[o],
                dst_ref=p1_ref.at[o],
                send_sem=p1_send.at[o],
                recv_sem=p1_recv.at[o],
                device_id=(e,),
                device_id_type=pl.DeviceIdType.MESH,
            )
            rdma.start()
            p1_rdmas.append(rdma)
        gather_chunk(my, 0)

        acc = p0_ref[0]
        for o in range(1, N_DEV):
            p1_rdmas[o - 1].wait_recv()
            acc = acc + p1_ref[o]
        red_ref[...] = acc

        p2_rdmas = []
        for o in range(1, N_DEV):
            e = jnp.mod(my + o, N_DEV)
            rdma = pltpu.make_async_remote_copy(
                src_ref=red_ref,
                dst_ref=p2_ref.at[o],
                send_sem=p2_send.at[o],
                recv_sem=p2_recv.at[o],
                device_id=(e,),
                device_id_type=pl.DeviceIdType.MESH,
            )
            rdma.start()
            p2_rdmas.append(rdma)

        out_ref[pl.ds(my * CHUNK, CHUNK), :] = red_ref[...].astype(jnp.float32)

        for r in p1_rdmas:
            r.wait_send()

        for o in range(1, N_DEV):
            p2_rdmas[o - 1].wait_recv()
            c = jnp.mod(my - o, N_DEV)
            out_ref[pl.ds(c * CHUNK, CHUNK), :] = p2_ref[o].astype(jnp.float32)
        for r in p2_rdmas:
            r.wait_send()

    return pl.pallas_call(
        body,
        out_shape=jax.ShapeDtypeStruct((N_IDX, D), jnp.float32),
        in_specs=[
            pl.BlockSpec(memory_space=pltpu.VMEM),
            pl.BlockSpec(memory_space=pltpu.VMEM),
        ],
        out_specs=pl.BlockSpec(memory_space=pltpu.VMEM),
        scratch_shapes=[
            pltpu.VMEM((V_PER, D), jnp.bfloat16),
            pltpu.VMEM((N_DEV, CHUNK, D), jnp.bfloat16),
            pltpu.VMEM((N_DEV, CHUNK, D), jnp.bfloat16),
            pltpu.VMEM((CHUNK, D), jnp.bfloat16),
            pltpu.VMEM((N_DEV, CHUNK, D), jnp.bfloat16),
            pltpu.SemaphoreType.DMA((N_DEV,)),
            pltpu.SemaphoreType.DMA((N_DEV,)),
            pltpu.SemaphoreType.DMA((N_DEV,)),
            pltpu.SemaphoreType.DMA((N_DEV,)),
        ],
        compiler_params=pltpu.CompilerParams(collective_id=0),
    )(idx2, table)


# device time: 35573 ns/iter; 1.0619x vs baseline; 1.0619x over previous
import jax
import jax.numpy as jnp
from jax import lax
from jax.experimental import pallas as pl
from jax.experimental.pallas import tpu as pltpu

N_DEV = 16
N_IDX = 1024
D = 512
V_PER = 4096
CHUNK = N_IDX // N_DEV


def kernel(table, idx):
    idx2 = idx.reshape(N_DEV, CHUNK)

    def body(idx_ref, table_ref, out_ref, tblb_ref, p0_ref, p1_ref, red_ref,
             p2_ref, p1_send, p1_recv, p2_send, p2_recv):
        my = lax.axis_index("i")

        barrier_sem = pltpu.get_barrier_semaphore()
        for o in range(1, N_DEV):
            pl.semaphore_signal(
                barrier_sem, inc=1,
                device_id=(jnp.mod(my + o, N_DEV),),
                device_id_type=pl.DeviceIdType.MESH,
            )
        pl.semaphore_wait(barrier_sem, N_DEV - 1)

        tblb_ref[...] = table_ref[...].astype(jnp.bfloat16)

        base = my * V_PER
        SLAB = 4
        col = lax.broadcasted_iota(jnp.int32, (CHUNK, V_PER), 1)

        def gather_slab(offsets):
            blocks = []
            for o in offsets:
                c = jnp.mod(my + o, N_DEV)
                loc = idx_ref[pl.ds(c, 1), :].reshape(CHUNK, 1) - base
                valid = jnp.logical_and(loc >= 0, loc < V_PER)
                blocks.append(
                    jnp.logical_and(col == loc, valid).astype(jnp.bfloat16))
            oh = jnp.concatenate(blocks, axis=0)
            part = jnp.dot(oh, tblb_ref[...],
                           preferred_element_type=jnp.float32)
            return part.astype(jnp.bfloat16)

        p1_rdmas = {}
        for s in range(N_DEV // SLAB):
            offsets = [(s * SLAB + j + 1) % N_DEV for j in range(SLAB)]
            part = gather_slab(offsets)
            for j, o in enumerate(offsets):
                p0_ref[o] = part[j * CHUNK:(j + 1) * CHUNK, :]
                if o == 0:
                    continue
                rdma = pltpu.make_async_remote_copy(
                    src_ref=p0_ref.at[o],
                    dst_ref=p1_ref.at[o],
                    send_sem=p1_send.at[o],
                    recv_sem=p1_recv.at[o],
                    device_id=(jnp.mod(my + o, N_DEV),),
                    device_id_type=pl.DeviceIdType.MESH,
                )
                rdma.start()
                p1_rdmas[o] = rdma
        p1_rdmas = [p1_rdmas[o] for o in range(1, N_DEV)]

        acc = p0_ref[0]
        for o in range(1, N_DEV):
            p1_rdmas[o - 1].wait_recv()
            acc = acc + p1_ref[o]
        red_ref[...] = acc

        p2_rdmas = []
        for o in range(1, N_DEV):
            e = jnp.mod(my + o, N_DEV)
            rdma = pltpu.make_async_remote_copy(
                src_ref=red_ref,
                dst_ref=p2_ref.at[o],
                send_sem=p2_send.at[o],
                recv_sem=p2_recv.at[o],
                device_id=(e,),
                device_id_type=pl.DeviceIdType.MESH,
            )
            rdma.start()
            p2_rdmas.append(rdma)

        out_ref[pl.ds(my * CHUNK, CHUNK), :] = red_ref[...].astype(jnp.float32)

        for r in p1_rdmas:
            r.wait_send()

        for o in range(1, N_DEV):
            p2_rdmas[o - 1].wait_recv()
            c = jnp.mod(my - o, N_DEV)
            out_ref[pl.ds(c * CHUNK, CHUNK), :] = p2_ref[o].astype(jnp.float32)
        for r in p2_rdmas:
            r.wait_send()

    return pl.pallas_call(
        body,
        out_shape=jax.ShapeDtypeStruct((N_IDX, D), jnp.float32),
        in_specs=[
            pl.BlockSpec(memory_space=pltpu.VMEM),
            pl.BlockSpec(memory_space=pltpu.VMEM),
        ],
        out_specs=pl.BlockSpec(memory_space=pltpu.VMEM),
        scratch_shapes=[
            pltpu.VMEM((V_PER, D), jnp.bfloat16),
            pltpu.VMEM((N_DEV, CHUNK, D), jnp.bfloat16),
            pltpu.VMEM((N_DEV, CHUNK, D), jnp.bfloat16),
            pltpu.VMEM((CHUNK, D), jnp.bfloat16),
            pltpu.VMEM((N_DEV, CHUNK, D), jnp.bfloat16),
            pltpu.SemaphoreType.DMA((N_DEV,)),
            pltpu.SemaphoreType.DMA((N_DEV,)),
            pltpu.SemaphoreType.DMA((N_DEV,)),
            pltpu.SemaphoreType.DMA((N_DEV,)),
        ],
        compiler_params=pltpu.CompilerParams(collective_id=0),
    )(idx2, table)


# device time: 34137 ns/iter; 1.1066x vs baseline; 1.0421x over previous
import jax
import jax.numpy as jnp
from jax import lax
from jax.experimental import pallas as pl
from jax.experimental.pallas import tpu as pltpu

N_DEV = 16
N_IDX = 1024
D = 512
V_PER = 4096
CHUNK = N_IDX // N_DEV


def kernel(table, idx):
    idx2 = idx.reshape(N_DEV, CHUNK)

    def body(idx_ref, table_ref, out_ref, tblb_ref, p0_ref, p1_ref, red_ref,
             p2_ref, p1_send, p1_recv, p2_send, p2_recv):
        my = lax.axis_index("i")

        barrier_sem = pltpu.get_barrier_semaphore()
        for o in range(1, N_DEV):
            pl.semaphore_signal(
                barrier_sem, inc=1,
                device_id=(jnp.mod(my + o, N_DEV),),
                device_id_type=pl.DeviceIdType.MESH,
            )

        tblb_ref[...] = table_ref[...].astype(jnp.bfloat16)

        base = my * V_PER
        SLAB = 4
        col = lax.broadcasted_iota(jnp.int32, (CHUNK, V_PER), 1)

        def gather_slab(offsets):
            blocks = []
            for o in offsets:
                c = jnp.mod(my + o, N_DEV)
                loc = idx_ref[pl.ds(c, 1), :].reshape(CHUNK, 1) - base
                blocks.append((col == loc).astype(jnp.bfloat16))
            oh = jnp.concatenate(blocks, axis=0)
            part = jnp.dot(oh, tblb_ref[...],
                           preferred_element_type=jnp.float32)
            return part.astype(jnp.bfloat16)

        p1_rdmas = {}
        for s in range(N_DEV // SLAB):
            offsets = [(s * SLAB + j + 1) % N_DEV for j in range(SLAB)]
            part = gather_slab(offsets)
            if s == 0:
                pl.semaphore_wait(barrier_sem, N_DEV - 1)
            for j, o in enumerate(offsets):
                p0_ref[o] = part[j * CHUNK:(j + 1) * CHUNK, :]
                if o == 0:
                    continue
                rdma = pltpu.make_async_remote_copy(
                    src_ref=p0_ref.at[o],
                    dst_ref=p1_ref.at[o],
                    send_sem=p1_send.at[o],
                    recv_sem=p1_recv.at[o],
                    device_id=(jnp.mod(my + o, N_DEV),),
                    device_id_type=pl.DeviceIdType.MESH,
                )
                rdma.start()
                p1_rdmas[o] = rdma
        p1_rdmas = [p1_rdmas[o] for o in range(1, N_DEV)]

        acc = p0_ref[0]
        for o in range(1, N_DEV):
            p1_rdmas[o - 1].wait_recv()
            acc = acc + p1_ref[o]
        red_ref[...] = acc

        p2_rdmas = []
        for o in range(1, N_DEV):
            e = jnp.mod(my + o, N_DEV)
            rdma = pltpu.make_async_remote_copy(
                src_ref=red_ref,
                dst_ref=p2_ref.at[o],
                send_sem=p2_send.at[o],
                recv_sem=p2_recv.at[o],
                device_id=(e,),
                device_id_type=pl.DeviceIdType.MESH,
            )
            rdma.start()
            p2_rdmas.append(rdma)

        out_ref[pl.ds(my * CHUNK, CHUNK), :] = red_ref[...].astype(jnp.float32)

        for r in p1_rdmas:
            r.wait_send()

        for o in range(1, N_DEV):
            p2_rdmas[o - 1].wait_recv()
            c = jnp.mod(my - o, N_DEV)
            out_ref[pl.ds(c * CHUNK, CHUNK), :] = p2_ref[o].astype(jnp.float32)
        for r in p2_rdmas:
            r.wait_send()

    return pl.pallas_call(
        body,
        out_shape=jax.ShapeDtypeStruct((N_IDX, D), jnp.float32),
        in_specs=[
            pl.BlockSpec(memory_space=pltpu.VMEM),
            pl.BlockSpec(memory_space=pltpu.VMEM),
        ],
        out_specs=pl.BlockSpec(memory_space=pltpu.VMEM),
        scratch_shapes=[
            pltpu.VMEM((V_PER, D), jnp.bfloat16),
            pltpu.VMEM((N_DEV, CHUNK, D), jnp.bfloat16),
            pltpu.VMEM((N_DEV, CHUNK, D), jnp.bfloat16),
            pltpu.VMEM((CHUNK, D), jnp.bfloat16),
            pltpu.VMEM((N_DEV, CHUNK, D), jnp.bfloat16),
            pltpu.SemaphoreType.DMA((N_DEV,)),
            pltpu.SemaphoreType.DMA((N_DEV,)),
            pltpu.SemaphoreType.DMA((N_DEV,)),
            pltpu.SemaphoreType.DMA((N_DEV,)),
        ],
        compiler_params=pltpu.CompilerParams(collective_id=0),
    )(idx2, table)
